# baseline (device time: 86513 ns/iter reference)
import jax
import jax.numpy as jnp
from jax import lax
from jax.experimental import pallas as pl
from jax.experimental.pallas import tpu as pltpu

N_DEV = 8
SQ = 256
SKV = 4096
H_LOC = 8
DH = 128
D_MODEL = 1024
HD = H_LOC * DH
BLK = 64
SCALE = 0.08838834764831843


def kernel(x, Wq, K_ext, V_ext, Wo):

    def body(x_ref, wq_ref, k_ref, v_ref, wo_ref, out_ref,
             wq_v, wo_v, kbuf, vbuf, wbuf, rs0, rs1, rs2,
             wq_sem, wo_sem, ksem, vsem, rs_send, rs_recv, ag_send, ag_recv):
        my = lax.axis_index("i")
        v = my ^ ((my >> 1) & 1)
        partners = [(v ^ (1 << r)) ^ (((v ^ (1 << r)) >> 1) & 1)
                    for r in range(3)]

        cwq = pltpu.make_async_copy(
            wq_ref.at[:, pl.ds(my * HD, HD)], wq_v, wq_sem)
        cwo = pltpu.make_async_copy(
            wo_ref.at[pl.ds(my * HD, HD), :], wo_v, wo_sem)
        cwq.start()
        cwo.start()

        ck = pltpu.make_async_copy(k_ref.at[0], kbuf, ksem)
        cv = pltpu.make_async_copy(v_ref.at[0], vbuf, vsem)
        ck.start()
        cv.start()

        barrier = pltpu.get_barrier_semaphore()
        for nbr in partners:
            pl.semaphore_signal(
                barrier, inc=1,
                device_id=(nbr,), device_id_type=pl.DeviceIdType.MESH,
            )
        pl.semaphore_wait(barrier, 3)

        qb = lax.broadcasted_iota(jnp.int32, (SQ, SKV), 0) // BLK
        kb = lax.broadcasted_iota(jnp.int32, (SQ, SKV), 1) // BLK
        mask = (qb == kb) | (kb == 0) | (((qb + kb) % 3) == 0)

        cwq.wait()
        q = jnp.dot(x_ref[0], wq_v[:, :], preferred_element_type=jnp.float32)

        ck.wait()
        cv.wait()
        cwo.wait()
        partial = jnp.zeros((SQ, HD), jnp.float32)
        for h in range(H_LOC):
            q_h = q[:, h * DH:(h + 1) * DH]
            s = lax.dot_general(
                q_h, kbuf[:, h, :],
                (((1,), (1,)), ((), ())),
                preferred_element_type=jnp.float32,
            ) * SCALE
            s = jnp.where(mask, s, -1e9)
            m = jnp.max(s, axis=1, keepdims=True)
            w = jnp.exp(s - m)
            denom = jnp.sum(w, axis=1, keepdims=True)
            ctx = jnp.dot(w, vbuf[:, h, :],
                          preferred_element_type=jnp.float32) / denom
            partial = partial + jnp.dot(
                ctx, wo_v[h * DH:(h + 1) * DH, :],
                preferred_element_type=jnp.float32)

        wbuf[:, :] = partial

        rsbufs = (rs0, rs1, rs2)
        base = my * 0
        for r, half in enumerate((SQ // 2, SQ // 4, SQ // 8)):
            bit = (v >> r) & 1
            send_off = base + (1 - bit) * half
            keep_off = base + bit * half
            rdma = pltpu.make_async_remote_copy(
                src_ref=wbuf.at[pl.ds(send_off, half), :],
                dst_ref=rsbufs[r],
                send_sem=rs_send.at[r],
                recv_sem=rs_recv.at[r],
                device_id=(partners[r],),
                device_id_type=pl.DeviceIdType.MESH,
            )
            rdma.start()
            rdma.wait()
            wbuf[pl.ds(keep_off, half), :] = (
                wbuf[pl.ds(keep_off, half), :] + rsbufs[r][:, :])
            base = keep_off

        for idx, (r, blk) in enumerate(((2, SQ // 8), (1, SQ // 4), (0, SQ // 2))):
            bit = (v >> r) & 1
            rdma = pltpu.make_async_remote_copy(
                src_ref=wbuf.at[pl.ds(base, blk), :],
                dst_ref=wbuf.at[pl.ds(base, blk), :],
                send_sem=ag_send.at[idx],
                recv_sem=ag_recv.at[idx],
                device_id=(partners[r],),
                device_id_type=pl.DeviceIdType.MESH,
            )
            rdma.start()
            rdma.wait()
            base = base - bit * blk

        out_ref[0, :, :] = wbuf[:, :]

    return pl.pallas_call(
        body,
        out_shape=jax.ShapeDtypeStruct((1, SQ, D_MODEL), jnp.float32),
        in_specs=[
            pl.BlockSpec(memory_space=pltpu.VMEM),
            pl.BlockSpec(memory_space=pl.ANY),
            pl.BlockSpec(memory_space=pl.ANY),
            pl.BlockSpec(memory_space=pl.ANY),
            pl.BlockSpec(memory_space=pl.ANY),
        ],
        out_specs=pl.BlockSpec(memory_space=pltpu.VMEM),
        scratch_shapes=[
            pltpu.VMEM((D_MODEL, HD), jnp.float32),
            pltpu.VMEM((HD, D_MODEL), jnp.float32),
            pltpu.VMEM((SKV, H_LOC, DH), jnp.float32),
            pltpu.VMEM((SKV, H_LOC, DH), jnp.float32),
            pltpu.VMEM((SQ, D_MODEL), jnp.float32),
            pltpu.VMEM((SQ // 2, D_MODEL), jnp.float32),
            pltpu.VMEM((SQ // 4, D_MODEL), jnp.float32),
            pltpu.VMEM((SQ // 8, D_MODEL), jnp.float32),
            pltpu.SemaphoreType.DMA,
            pltpu.SemaphoreType.DMA,
            pltpu.SemaphoreType.DMA,
            pltpu.SemaphoreType.DMA,
            pltpu.SemaphoreType.DMA((3,)),
            pltpu.SemaphoreType.DMA((3,)),
            pltpu.SemaphoreType.DMA((3,)),
            pltpu.SemaphoreType.DMA((3,)),
        ],
        compiler_params=pltpu.CompilerParams(
            collective_id=0,
            vmem_limit_bytes=64 * 1024 * 1024,
        ),
    )(x, Wq, K_ext, V_ext, Wo)


# device time: 56802 ns/iter; 1.5231x vs baseline; 1.5231x over previous
import os

import jax
import jax.numpy as jnp
from jax import lax

_SKIP_AR = os.environ.get("KERNEL_SKIP_AR") == "1"
_SKIP_ATTN = os.environ.get("KERNEL_SKIP_ATTN") == "1"
from jax.experimental import pallas as pl
from jax.experimental.pallas import tpu as pltpu

N_DEV = 8
SQ = 256
SKV = 4096
H_LOC = 8
DH = 128
D_MODEL = 1024
HD = H_LOC * DH
BLK = 64
SCALE = 0.08838834764831843


def kernel(x, Wq, K_ext, V_ext, Wo):

    def body(x_ref, wq_ref, k_ref, v_ref, wo_ref, out_ref,
             wq_v, wo_v, kbuf, vbuf, wbuf, rs0, rs1, rs2,
             wq_sem, wo_sem, ksem, vsem, rs_send, rs_recv, ag_send, ag_recv):
        my = lax.axis_index("i")
        v = my ^ ((my >> 1) & 1)
        partners = [(v ^ (1 << r)) ^ (((v ^ (1 << r)) >> 1) & 1)
                    for r in range(3)]

        cwq = pltpu.make_async_copy(
            wq_ref.at[:, pl.ds(my * HD, HD)], wq_v, wq_sem)
        cwo = pltpu.make_async_copy(
            wo_ref.at[pl.ds(my * HD, HD), :], wo_v, wo_sem)
        cwq.start()
        cwo.start()

        def kv_copies(h):
            ck = pltpu.make_async_copy(
                k_ref.at[0, :, h, :], kbuf.at[h], ksem.at[h])
            cv = pltpu.make_async_copy(
                v_ref.at[0, :, h, :], vbuf.at[h], vsem.at[h])
            return ck, cv

        pending = {}
        if not _SKIP_ATTN:
            for h in range(H_LOC):
                pending[h] = kv_copies(h)
                pending[h][0].start()
                pending[h][1].start()

        barrier = pltpu.get_barrier_semaphore()
        for nbr in partners:
            pl.semaphore_signal(
                barrier, inc=1,
                device_id=(nbr,), device_id_type=pl.DeviceIdType.MESH,
            )
        pl.semaphore_wait(barrier, 3)

        qb = lax.broadcasted_iota(jnp.int32, (SQ, SKV), 0) // BLK
        kb = lax.broadcasted_iota(jnp.int32, (SQ, SKV), 1) // BLK
        mask = (qb == kb) | (kb == 0) | (((qb + kb) % 3) == 0)

        cwq.wait()
        q = jnp.dot(x_ref[0], wq_v[:, :], preferred_element_type=jnp.float32)
        if _SKIP_ATTN:
            cwo.wait()

        partial = jnp.zeros((SQ, HD), jnp.float32)
        if _SKIP_ATTN:
            partial = q
        for h in range(0 if _SKIP_ATTN else H_LOC):
            ck, cv = pending.pop(h)
            ck.wait()
            cv.wait()
            if h == 0:
                cwo.wait()
            q_h = q[:, h * DH:(h + 1) * DH]
            s = lax.dot_general(
                q_h, kbuf[h],
                (((1,), (1,)), ((), ())),
                preferred_element_type=jnp.float32,
            ) * SCALE
            w = jnp.exp(jnp.where(mask, s, -1e9))
            denom = jnp.sum(w, axis=1, keepdims=True)
            ctx = jnp.dot(w, vbuf[h],
                          preferred_element_type=jnp.float32) / denom
            partial = partial + jnp.dot(
                ctx, wo_v[h * DH:(h + 1) * DH, :],
                preferred_element_type=jnp.float32)

        wbuf[:, :] = partial.astype(jnp.bfloat16)
        if _SKIP_AR:
            out_ref[0, :, :] = wbuf[:, :].astype(jnp.float32)
            return

        rsbufs = (rs0, rs1, rs2)
        base = my * 0
        for r, half in enumerate((SQ // 2, SQ // 4, SQ // 8)):
            bit = (v >> r) & 1
            send_off = base + (1 - bit) * half
            keep_off = base + bit * half
            rdma = pltpu.make_async_remote_copy(
                src_ref=wbuf.at[pl.ds(send_off, half), :],
                dst_ref=rsbufs[r],
                send_sem=rs_send.at[r],
                recv_sem=rs_recv.at[r],
                device_id=(partners[r],),
                device_id_type=pl.DeviceIdType.MESH,
            )
            rdma.start()
            rdma.wait()
            wbuf[pl.ds(keep_off, half), :] = (
                wbuf[pl.ds(keep_off, half), :].astype(jnp.float32)
                + rsbufs[r][:, :].astype(jnp.float32)
            ).astype(jnp.bfloat16)
            base = keep_off

        for idx, (r, blk) in enumerate(((2, SQ // 8), (1, SQ // 4), (0, SQ // 2))):
            bit = (v >> r) & 1
            rdma = pltpu.make_async_remote_copy(
                src_ref=wbuf.at[pl.ds(base, blk), :],
                dst_ref=wbuf.at[pl.ds(base, blk), :],
                send_sem=ag_send.at[idx],
                recv_sem=ag_recv.at[idx],
                device_id=(partners[r],),
                device_id_type=pl.DeviceIdType.MESH,
            )
            rdma.start()
            rdma.wait()
            base = base - bit * blk

        out_ref[0, :, :] = wbuf[:, :].astype(jnp.float32)

    return pl.pallas_call(
        body,
        out_shape=jax.ShapeDtypeStruct((1, SQ, D_MODEL), jnp.float32),
        in_specs=[
            pl.BlockSpec(memory_space=pltpu.VMEM),
            pl.BlockSpec(memory_space=pl.ANY),
            pl.BlockSpec(memory_space=pl.ANY),
            pl.BlockSpec(memory_space=pl.ANY),
            pl.BlockSpec(memory_space=pl.ANY),
        ],
        out_specs=pl.BlockSpec(memory_space=pltpu.VMEM),
        scratch_shapes=[
            pltpu.VMEM((D_MODEL, HD), jnp.float32),
            pltpu.VMEM((HD, D_MODEL), jnp.float32),
            pltpu.VMEM((H_LOC, SKV, DH), jnp.float32),
            pltpu.VMEM((H_LOC, SKV, DH), jnp.float32),
            pltpu.VMEM((SQ, D_MODEL), jnp.bfloat16),
            pltpu.VMEM((SQ // 2, D_MODEL), jnp.bfloat16),
            pltpu.VMEM((SQ // 4, D_MODEL), jnp.bfloat16),
            pltpu.VMEM((SQ // 8, D_MODEL), jnp.bfloat16),
            pltpu.SemaphoreType.DMA,
            pltpu.SemaphoreType.DMA,
            pltpu.SemaphoreType.DMA((H_LOC,)),
            pltpu.SemaphoreType.DMA((H_LOC,)),
            pltpu.SemaphoreType.DMA((3,)),
            pltpu.SemaphoreType.DMA((3,)),
            pltpu.SemaphoreType.DMA((3,)),
            pltpu.SemaphoreType.DMA((3,)),
        ],
        compiler_params=pltpu.CompilerParams(
            collective_id=0,
            vmem_limit_bytes=64 * 1024 * 1024,
        ),
    )(x, Wq, K_ext, V_ext, Wo)


# device time: 49118 ns/iter; 1.7613x vs baseline; 1.1564x over previous
import os

import jax
import jax.numpy as jnp
from jax import lax

_SKIP_AR = os.environ.get("KERNEL_SKIP_AR") == "1"
_SKIP_ATTN = os.environ.get("KERNEL_SKIP_ATTN") == "1"
from jax.experimental import pallas as pl
from jax.experimental.pallas import tpu as pltpu

N_DEV = 8
SQ = 256
SKV = 4096
H_LOC = 8
DH = 128
D_MODEL = 1024
HD = H_LOC * DH
BLK = 64
SCALE = 0.08838834764831843


def kernel(x, Wq, K_ext, V_ext, Wo):

    def body(x_ref, wq_ref, k_ref, v_ref, wo_ref, out_ref,
             wq_v, wo_v, kbuf, vbuf, wbuf, rs_slots,
             wq_sem, wo_sem, ksem, vsem, rs_send, rs_recv, ag_send, ag_recv):
        my = lax.axis_index("i")
        v = my ^ ((my >> 1) & 1)
        peers = [(v ^ j) ^ (((v ^ j) >> 1) & 1) for j in range(1, N_DEV)]

        cwq = pltpu.make_async_copy(
            wq_ref.at[:, pl.ds(my * HD, HD)], wq_v, wq_sem)
        cwo = pltpu.make_async_copy(
            wo_ref.at[pl.ds(my * HD, HD), :], wo_v, wo_sem)
        cwq.start()
        cwo.start()

        def kv_copies(h):
            ck = pltpu.make_async_copy(
                k_ref.at[0, :, h, :], kbuf.at[h], ksem.at[h])
            cv = pltpu.make_async_copy(
                v_ref.at[0, :, h, :], vbuf.at[h], vsem.at[h])
            return ck, cv

        pending = {}
        if not _SKIP_ATTN:
            for h in range(H_LOC):
                pending[h] = kv_copies(h)
                pending[h][0].start()
                pending[h][1].start()

        barrier = pltpu.get_barrier_semaphore()
        for nbr in peers:
            pl.semaphore_signal(
                barrier, inc=1,
                device_id=(nbr,), device_id_type=pl.DeviceIdType.MESH,
            )
        pl.semaphore_wait(barrier, N_DEV - 1)

        qb = lax.broadcasted_iota(jnp.int32, (SQ, SKV), 0) // BLK
        kb = lax.broadcasted_iota(jnp.int32, (SQ, SKV), 1) // BLK
        mask = (qb == kb) | (kb == 0) | (((qb + kb) % 3) == 0)

        cwq.wait()
        q = jnp.dot(x_ref[0], wq_v[:, :], preferred_element_type=jnp.float32)
        if _SKIP_ATTN:
            cwo.wait()

        partial = jnp.zeros((SQ, HD), jnp.float32)
        if _SKIP_ATTN:
            partial = q
        for h in range(0 if _SKIP_ATTN else H_LOC):
            ck, cv = pending.pop(h)
            ck.wait()
            cv.wait()
            if h == 0:
                cwo.wait()
            q_h = q[:, h * DH:(h + 1) * DH]
            s = lax.dot_general(
                q_h, kbuf[h],
                (((1,), (1,)), ((), ())),
                preferred_element_type=jnp.float32,
            ) * SCALE
            w = jnp.exp(jnp.where(mask, s, -1e9))
            denom = jnp.sum(w, axis=1, keepdims=True)
            ctx = jnp.dot(w, vbuf[h],
                          preferred_element_type=jnp.float32) / denom
            partial = partial + jnp.dot(
                ctx, wo_v[h * DH:(h + 1) * DH, :],
                preferred_element_type=jnp.float32)

        wbuf[:, :] = partial.astype(jnp.bfloat16)
        if _SKIP_AR:
            out_ref[0, :, :] = wbuf[:, :].astype(jnp.float32)
            return

        blk = SQ // N_DEV
        rs_rdmas = []
        for j in range(1, N_DEV):
            t_v = v ^ j
            rdma = pltpu.make_async_remote_copy(
                src_ref=wbuf.at[pl.ds(t_v * blk, blk), :],
                dst_ref=rs_slots.at[j - 1],
                send_sem=rs_send.at[j - 1],
                recv_sem=rs_recv.at[j - 1],
                device_id=(peers[j - 1],),
                device_id_type=pl.DeviceIdType.MESH,
            )
            rdma.start()
            rs_rdmas.append(rdma)
        acc = wbuf[pl.ds(v * blk, blk), :].astype(jnp.float32)
        for j in range(1, N_DEV):
            rs_rdmas[j - 1].wait()
            acc = acc + rs_slots[j - 1].astype(jnp.float32)
        wbuf[pl.ds(v * blk, blk), :] = acc.astype(jnp.bfloat16)

        ag_rdmas = []
        for j in range(1, N_DEV):
            rdma = pltpu.make_async_remote_copy(
                src_ref=wbuf.at[pl.ds(v * blk, blk), :],
                dst_ref=wbuf.at[pl.ds(v * blk, blk), :],
                send_sem=ag_send.at[j - 1],
                recv_sem=ag_recv.at[j - 1],
                device_id=(peers[j - 1],),
                device_id_type=pl.DeviceIdType.MESH,
            )
            rdma.start()
            ag_rdmas.append(rdma)
        for j in range(1, N_DEV):
            ag_rdmas[j - 1].wait()

        out_ref[0, :, :] = wbuf[:, :].astype(jnp.float32)

    return pl.pallas_call(
        body,
        out_shape=jax.ShapeDtypeStruct((1, SQ, D_MODEL), jnp.float32),
        in_specs=[
            pl.BlockSpec(memory_space=pltpu.VMEM),
            pl.BlockSpec(memory_space=pl.ANY),
            pl.BlockSpec(memory_space=pl.ANY),
            pl.BlockSpec(memory_space=pl.ANY),
            pl.BlockSpec(memory_space=pl.ANY),
        ],
        out_specs=pl.BlockSpec(memory_space=pltpu.VMEM),
        scratch_shapes=[
            pltpu.VMEM((D_MODEL, HD), jnp.float32),
            pltpu.VMEM((HD, D_MODEL), jnp.float32),
            pltpu.VMEM((H_LOC, SKV, DH), jnp.float32),
            pltpu.VMEM((H_LOC, SKV, DH), jnp.float32),
            pltpu.VMEM((SQ, D_MODEL), jnp.bfloat16),
            pltpu.VMEM((N_DEV - 1, SQ // N_DEV, D_MODEL), jnp.bfloat16),
            pltpu.SemaphoreType.DMA,
            pltpu.SemaphoreType.DMA,
            pltpu.SemaphoreType.DMA((H_LOC,)),
            pltpu.SemaphoreType.DMA((H_LOC,)),
            pltpu.SemaphoreType.DMA((N_DEV - 1,)),
            pltpu.SemaphoreType.DMA((N_DEV - 1,)),
            pltpu.SemaphoreType.DMA((N_DEV - 1,)),
            pltpu.SemaphoreType.DMA((N_DEV - 1,)),
        ],
        compiler_params=pltpu.CompilerParams(
            collective_id=0,
            vmem_limit_bytes=64 * 1024 * 1024,
        ),
    )(x, Wq, K_ext, V_ext, Wo)


# device time: 46779 ns/iter; 1.8494x vs baseline; 1.0500x over previous
import os

import jax
import jax.numpy as jnp
from jax import lax

_SKIP_AR = os.environ.get("KERNEL_SKIP_AR") == "1"
_SKIP_ATTN = os.environ.get("KERNEL_SKIP_ATTN") == "1"
from jax.experimental import pallas as pl
from jax.experimental.pallas import tpu as pltpu

N_DEV = 8
SQ = 256
SKV = 4096
H_LOC = 8
DH = 128
D_MODEL = 1024
HD = H_LOC * DH
BLK = 64
SCALE = 0.08838834764831843


def kernel(x, Wq, K_ext, V_ext, Wo):

    def body(x_ref, wq_ref, k_ref, v_ref, wo_ref, out_ref,
             wq_v, wo_v, kbuf, vbuf, wbuf, rs_slots,
             wq_sem, wo_sem, ksem, vsem, rs_send, rs_recv, ag_send, ag_recv):
        my = lax.axis_index("i")
        v = my ^ ((my >> 1) & 1)
        peers = [(v ^ j) ^ (((v ^ j) >> 1) & 1) for j in range(1, N_DEV)]

        cwq = pltpu.make_async_copy(
            wq_ref.at[:, pl.ds(my * HD, HD)], wq_v, wq_sem)
        cwo = pltpu.make_async_copy(
            wo_ref.at[pl.ds(my * HD, HD), :], wo_v, wo_sem)
        cwq.start()
        cwo.start()

        def kv_copies(h):
            ck = pltpu.make_async_copy(
                k_ref.at[0, :, h, :], kbuf.at[h], ksem.at[h])
            cv = pltpu.make_async_copy(
                v_ref.at[0, :, h, :], vbuf.at[h], vsem.at[h])
            return ck, cv

        pending = {}
        if not _SKIP_ATTN:
            for h in range(H_LOC):
                pending[h] = kv_copies(h)
                pending[h][0].start()
                pending[h][1].start()

        barrier = pltpu.get_barrier_semaphore()
        for nbr in peers:
            pl.semaphore_signal(
                barrier, inc=1,
                device_id=(nbr,), device_id_type=pl.DeviceIdType.MESH,
            )
        pl.semaphore_wait(barrier, N_DEV - 1)

        qb = lax.broadcasted_iota(jnp.int32, (SQ, SKV), 0) // BLK
        kb = lax.broadcasted_iota(jnp.int32, (SQ, SKV), 1) // BLK
        mask = (qb == kb) | (kb == 0) | (((qb + kb) % 3) == 0)

        cwq.wait()
        q = jnp.dot(x_ref[0], wq_v[:, :],
                    preferred_element_type=jnp.float32) * SCALE
        if _SKIP_ATTN:
            cwo.wait()

        partial = jnp.zeros((SQ, HD), jnp.float32)
        if _SKIP_ATTN:
            partial = q
        for h in range(0 if _SKIP_ATTN else H_LOC):
            ck, cv = pending.pop(h)
            ck.wait()
            cv.wait()
            if h == 0:
                cwo.wait()
            q_h = q[:, h * DH:(h + 1) * DH]
            s = lax.dot_general(
                q_h, kbuf[h],
                (((1,), (1,)), ((), ())),
                preferred_element_type=jnp.float32,
            )
            w = jnp.exp(jnp.where(mask, s, -1e9).astype(jnp.bfloat16))
            denom = jnp.sum(w.astype(jnp.float32), axis=1, keepdims=True)
            ctx = jnp.dot(w, vbuf[h],
                          preferred_element_type=jnp.float32) / denom
            partial = partial + jnp.dot(
                ctx, wo_v[h * DH:(h + 1) * DH, :],
                preferred_element_type=jnp.float32)

        wbuf[:, :] = partial.astype(jnp.bfloat16)
        if _SKIP_AR:
            out_ref[0, :, :] = wbuf[:, :].astype(jnp.float32)
            return

        blk = SQ // N_DEV
        rs_rdmas = []
        for j in range(1, N_DEV):
            t_v = v ^ j
            rdma = pltpu.make_async_remote_copy(
                src_ref=wbuf.at[pl.ds(t_v * blk, blk), :],
                dst_ref=rs_slots.at[j - 1],
                send_sem=rs_send.at[j - 1],
                recv_sem=rs_recv.at[j - 1],
                device_id=(peers[j - 1],),
                device_id_type=pl.DeviceIdType.MESH,
            )
            rdma.start()
            rs_rdmas.append(rdma)
        acc = wbuf[pl.ds(v * blk, blk), :].astype(jnp.float32)
        for j in range(1, N_DEV):
            rs_rdmas[j - 1].wait()
            acc = acc + rs_slots[j - 1].astype(jnp.float32)
        wbuf[pl.ds(v * blk, blk), :] = acc.astype(jnp.bfloat16)

        ag_rdmas = []
        for j in range(1, N_DEV):
            rdma = pltpu.make_async_remote_copy(
                src_ref=wbuf.at[pl.ds(v * blk, blk), :],
                dst_ref=wbuf.at[pl.ds(v * blk, blk), :],
                send_sem=ag_send.at[j - 1],
                recv_sem=ag_recv.at[j - 1],
                device_id=(peers[j - 1],),
                device_id_type=pl.DeviceIdType.MESH,
            )
            rdma.start()
            ag_rdmas.append(rdma)
        for j in range(1, N_DEV):
            ag_rdmas[j - 1].wait()

        out_ref[0, :, :] = wbuf[:, :].astype(jnp.float32)

    return pl.pallas_call(
        body,
        out_shape=jax.ShapeDtypeStruct((1, SQ, D_MODEL), jnp.float32),
        in_specs=[
            pl.BlockSpec(memory_space=pltpu.VMEM),
            pl.BlockSpec(memory_space=pl.ANY),
            pl.BlockSpec(memory_space=pl.ANY),
            pl.BlockSpec(memory_space=pl.ANY),
            pl.BlockSpec(memory_space=pl.ANY),
        ],
        out_specs=pl.BlockSpec(memory_space=pltpu.VMEM),
        scratch_shapes=[
            pltpu.VMEM((D_MODEL, HD), jnp.float32),
            pltpu.VMEM((HD, D_MODEL), jnp.float32),
            pltpu.VMEM((H_LOC, SKV, DH), jnp.float32),
            pltpu.VMEM((H_LOC, SKV, DH), jnp.float32),
            pltpu.VMEM((SQ, D_MODEL), jnp.bfloat16),
            pltpu.VMEM((N_DEV - 1, SQ // N_DEV, D_MODEL), jnp.bfloat16),
            pltpu.SemaphoreType.DMA,
            pltpu.SemaphoreType.DMA,
            pltpu.SemaphoreType.DMA((H_LOC,)),
            pltpu.SemaphoreType.DMA((H_LOC,)),
            pltpu.SemaphoreType.DMA((N_DEV - 1,)),
            pltpu.SemaphoreType.DMA((N_DEV - 1,)),
            pltpu.SemaphoreType.DMA((N_DEV - 1,)),
            pltpu.SemaphoreType.DMA((N_DEV - 1,)),
        ],
        compiler_params=pltpu.CompilerParams(
            collective_id=0,
            vmem_limit_bytes=64 * 1024 * 1024,
        ),
    )(x, Wq, K_ext, V_ext, Wo)
